# baseline (speedup 1.0000x reference)
import functools

import jax
import jax.numpy as jnp
from jax.experimental import pallas as pl
from jax.experimental.pallas import tpu as pltpu


def _chamfer_body(x1_ref, x2t_ref, o_ref, acc_ref, *, n_i, scale1, scale2):
    b = pl.program_id(0)
    i = pl.program_id(1)

    a = x1_ref[0]
    bt = x2t_ref[0]

    sq1 = jnp.sum(a * a, axis=1, keepdims=True)
    sq2 = jnp.sum(bt * bt, axis=0, keepdims=True)
    aug_a = jnp.concatenate(
        [-2.0 * a, sq1, jnp.ones_like(sq1)], axis=1
    )
    aug_b = jnp.concatenate([bt, jnp.ones_like(sq2), sq2], axis=0)
    d = jax.lax.dot_general(
        aug_a,
        aug_b,
        (((1,), (0,)), ((), ())),
        preferred_element_type=jnp.float32,
        precision=jax.lax.Precision.HIGHEST,
    )

    rowmin = jnp.maximum(jnp.min(d, axis=1), 0.0)
    colmin = jnp.min(d, axis=0, keepdims=True)

    @pl.when(jnp.logical_and(b == 0, i == 0))
    def _():
        o_ref[0, 0] = 0.0

    @pl.when(i == 0)
    def _():
        acc_ref[0:1, :] = colmin

    @pl.when(i > 0)
    def _():
        acc_ref[0:1, :] = jnp.minimum(acc_ref[0:1, :], colmin)

    o_ref[0, 0] += jnp.sum(rowmin) * scale1

    @pl.when(i == n_i - 1)
    def _():
        o_ref[0, 0] += jnp.sum(jnp.maximum(acc_ref[0, :], 0.0)) * scale2


@jax.jit
def kernel(xyz1, xyz2):
    B, N, _ = xyz1.shape
    _, M, _ = xyz2.shape
    TN = 1024
    n_i = N // TN

    x2t = jnp.transpose(xyz2, (0, 2, 1))

    body = functools.partial(
        _chamfer_body,
        n_i=n_i,
        scale1=1.0 / (B * N),
        scale2=1.0 / (B * M),
    )

    out = pl.pallas_call(
        body,
        grid=(B, n_i),
        in_specs=[
            pl.BlockSpec((1, TN, 3), lambda b, i: (b, i, 0)),
            pl.BlockSpec((1, 3, M), lambda b, i: (b, 0, 0)),
        ],
        out_specs=pl.BlockSpec(
            (1, 1), lambda b, i: (0, 0), memory_space=pltpu.SMEM
        ),
        out_shape=jax.ShapeDtypeStruct((1, 1), jnp.float32),
        scratch_shapes=[pltpu.VMEM((1, M), jnp.float32)],
    )(xyz1, x2t)

    return out[0, 0]


# speedup vs baseline: 3.5651x; 3.5651x over previous
import functools

import jax
import jax.numpy as jnp
from jax.experimental import pallas as pl
from jax.experimental.pallas import tpu as pltpu


def _chamfer_body(x1_ref, x2t_ref, o_ref, acc_ref, *, n_i, scale1, scale2):
    b = pl.program_id(0)
    i = pl.program_id(1)

    a = x1_ref[0]
    bt = x2t_ref[0]

    sq1 = jnp.sum(a * a, axis=1, keepdims=True)
    sq2 = jnp.sum(bt * bt, axis=0, keepdims=True)
    inner = jax.lax.dot_general(
        a, bt, (((1,), (0,)), ((), ())), preferred_element_type=jnp.float32
    )
    d = sq1 + sq2 - 2.0 * inner

    rowmin = jnp.maximum(jnp.min(d, axis=1), 0.0)
    colmin = jnp.min(d, axis=0, keepdims=True)

    @pl.when(jnp.logical_and(b == 0, i == 0))
    def _():
        o_ref[0, 0] = 0.0

    @pl.when(i == 0)
    def _():
        acc_ref[0:1, :] = colmin

    @pl.when(i > 0)
    def _():
        acc_ref[0:1, :] = jnp.minimum(acc_ref[0:1, :], colmin)

    o_ref[0, 0] += jnp.sum(rowmin) * scale1

    @pl.when(i == n_i - 1)
    def _():
        o_ref[0, 0] += jnp.sum(jnp.maximum(acc_ref[0, :], 0.0)) * scale2


@jax.jit
def kernel(xyz1, xyz2):
    B, N, _ = xyz1.shape
    _, M, _ = xyz2.shape
    TN = 1024
    n_i = N // TN

    x2t = jnp.transpose(xyz2, (0, 2, 1))

    body = functools.partial(
        _chamfer_body,
        n_i=n_i,
        scale1=1.0 / (B * N),
        scale2=1.0 / (B * M),
    )

    out = pl.pallas_call(
        body,
        grid=(B, n_i),
        in_specs=[
            pl.BlockSpec((1, TN, 3), lambda b, i: (b, i, 0)),
            pl.BlockSpec((1, 3, M), lambda b, i: (b, 0, 0)),
        ],
        out_specs=pl.BlockSpec(
            (1, 1), lambda b, i: (0, 0), memory_space=pltpu.SMEM
        ),
        out_shape=jax.ShapeDtypeStruct((1, 1), jnp.float32),
        scratch_shapes=[pltpu.VMEM((1, M), jnp.float32)],
    )(xyz1, x2t)

    return out[0, 0]


# speedup vs baseline: 3.9466x; 1.1070x over previous
import functools

import jax
import jax.numpy as jnp
from jax.experimental import pallas as pl
from jax.experimental.pallas import tpu as pltpu


def _chamfer_body(x1_ref, x2t_ref, o_ref, acc_ref, *, n_i, scale1, scale2):
    b = pl.program_id(0)
    i = pl.program_id(1)

    a = x1_ref[0]
    bt = x2t_ref[0]

    sq1 = jnp.sum(a * a, axis=1, keepdims=True)
    sq2 = jnp.sum(bt * bt, axis=0, keepdims=True)
    inner2 = jax.lax.dot_general(
        -2.0 * a, bt, (((1,), (0,)), ((), ())), preferred_element_type=jnp.float32
    )
    d = (sq1 + sq2) + inner2

    rowmin = jnp.maximum(jnp.min(d, axis=1), 0.0)
    colmin = jnp.min(d, axis=0, keepdims=True)

    @pl.when(jnp.logical_and(b == 0, i == 0))
    def _():
        o_ref[0, 0] = 0.0

    @pl.when(i == 0)
    def _():
        acc_ref[0:1, :] = colmin

    @pl.when(i > 0)
    def _():
        acc_ref[0:1, :] = jnp.minimum(acc_ref[0:1, :], colmin)

    o_ref[0, 0] += jnp.sum(rowmin) * scale1

    @pl.when(i == n_i - 1)
    def _():
        o_ref[0, 0] += jnp.sum(jnp.maximum(acc_ref[0, :], 0.0)) * scale2


@jax.jit
def kernel(xyz1, xyz2):
    B, N, _ = xyz1.shape
    _, M, _ = xyz2.shape
    TN = 1024
    n_i = N // TN

    x2t = jnp.transpose(xyz2, (0, 2, 1))

    body = functools.partial(
        _chamfer_body,
        n_i=n_i,
        scale1=1.0 / (B * N),
        scale2=1.0 / (B * M),
    )

    out = pl.pallas_call(
        body,
        grid=(B, n_i),
        in_specs=[
            pl.BlockSpec((1, TN, 3), lambda b, i: (b, i, 0)),
            pl.BlockSpec((1, 3, M), lambda b, i: (b, 0, 0)),
        ],
        out_specs=pl.BlockSpec(
            (1, 1), lambda b, i: (0, 0), memory_space=pltpu.SMEM
        ),
        out_shape=jax.ShapeDtypeStruct((1, 1), jnp.float32),
        scratch_shapes=[pltpu.VMEM((1, M), jnp.float32)],
    )(xyz1, x2t)

    return out[0, 0]


# speedup vs baseline: 4.1959x; 1.0632x over previous
import functools

import jax
import jax.numpy as jnp
from jax.experimental import pallas as pl
from jax.experimental.pallas import tpu as pltpu


def _chamfer_body(x1_ref, x2t_ref, o_ref, acc_ref, *, n_i, scale1, scale2):
    b = pl.program_id(0)
    i = pl.program_id(1)

    a = x1_ref[0]
    bt = x2t_ref[0]

    sq1 = jnp.sum(a * a, axis=1, keepdims=True)
    sq2 = jnp.sum(bt * bt, axis=0, keepdims=True)
    inner2 = jax.lax.dot_general(
        -2.0 * a, bt, (((1,), (0,)), ((), ())), preferred_element_type=jnp.float32
    )
    d = (sq1 + sq2) + inner2

    rowmin = jnp.maximum(jnp.min(d, axis=1), 0.0)
    colmin = jnp.min(d, axis=0, keepdims=True)

    @pl.when(jnp.logical_and(b == 0, i == 0))
    def _():
        o_ref[0, 0] = 0.0

    @pl.when(i == 0)
    def _():
        acc_ref[0:1, :] = colmin

    @pl.when(i > 0)
    def _():
        acc_ref[0:1, :] = jnp.minimum(acc_ref[0:1, :], colmin)

    o_ref[0, 0] += jnp.sum(rowmin) * scale1

    @pl.when(i == n_i - 1)
    def _():
        o_ref[0, 0] += jnp.sum(jnp.maximum(acc_ref[0, :], 0.0)) * scale2


@jax.jit
def kernel(xyz1, xyz2):
    B, N, _ = xyz1.shape
    _, M, _ = xyz2.shape
    TN = 2048
    n_i = N // TN

    x2t = jnp.transpose(xyz2, (0, 2, 1))

    body = functools.partial(
        _chamfer_body,
        n_i=n_i,
        scale1=1.0 / (B * N),
        scale2=1.0 / (B * M),
    )

    out = pl.pallas_call(
        body,
        grid=(B, n_i),
        in_specs=[
            pl.BlockSpec((1, TN, 3), lambda b, i: (b, i, 0)),
            pl.BlockSpec((1, 3, M), lambda b, i: (b, 0, 0)),
        ],
        out_specs=pl.BlockSpec(
            (1, 1), lambda b, i: (0, 0), memory_space=pltpu.SMEM
        ),
        out_shape=jax.ShapeDtypeStruct((1, 1), jnp.float32),
        scratch_shapes=[pltpu.VMEM((1, M), jnp.float32)],
    )(xyz1, x2t)

    return out[0, 0]


# speedup vs baseline: 4.3818x; 1.0443x over previous
import functools

import jax
import jax.numpy as jnp
from jax.experimental import pallas as pl
from jax.experimental.pallas import tpu as pltpu


def _chamfer_body(x1_ref, x2t_ref, o_ref, *, mc, scale1, scale2):
    b = pl.program_id(0)

    a = x1_ref[0]
    bt = x2t_ref[0]
    m = bt.shape[1]
    n_chunks = m // mc

    sq1 = jnp.sum(a * a, axis=1, keepdims=True)
    a2 = -2.0 * a

    @pl.when(b == 0)
    def _():
        o_ref[0, 0] = 0.0

    rowmin = None
    colsum = jnp.float32(0.0)
    for c in range(n_chunks):
        btc = bt[:, c * mc:(c + 1) * mc]
        sq2c = jnp.sum(btc * btc, axis=0, keepdims=True)
        inner2 = jax.lax.dot_general(
            a2, btc, (((1,), (0,)), ((), ())),
            preferred_element_type=jnp.float32,
        )
        d = (sq1 + sq2c) + inner2
        rm_c = jnp.min(d, axis=1)
        rowmin = rm_c if rowmin is None else jnp.minimum(rowmin, rm_c)
        colsum += jnp.sum(jnp.maximum(jnp.min(d, axis=0), 0.0))

    rowsum = jnp.sum(jnp.maximum(rowmin, 0.0))
    o_ref[0, 0] += rowsum * scale1 + colsum * scale2


@jax.jit
def kernel(xyz1, xyz2):
    B, N, _ = xyz1.shape
    _, M, _ = xyz2.shape

    x2t = jnp.transpose(xyz2, (0, 2, 1))

    body = functools.partial(
        _chamfer_body,
        mc=1024,
        scale1=1.0 / (B * N),
        scale2=1.0 / (B * M),
    )

    out = pl.pallas_call(
        body,
        grid=(B,),
        in_specs=[
            pl.BlockSpec((1, N, 3), lambda b: (b, 0, 0)),
            pl.BlockSpec((1, 3, M), lambda b: (b, 0, 0)),
        ],
        out_specs=pl.BlockSpec(
            (1, 1), lambda b: (0, 0), memory_space=pltpu.SMEM
        ),
        out_shape=jax.ShapeDtypeStruct((1, 1), jnp.float32),
    )(xyz1, x2t)

    return out[0, 0]
